# Initial kernel scaffold; baseline (speedup 1.0000x reference)
import jax
import jax.numpy as jnp
from jax.experimental import pallas as pl


def kernel(z, codebooks, W1, b1, W2, b2):
    raise NotImplementedError("write your pallas kernel here")


# speedup vs baseline: 1.0976x; 1.0976x over previous
import jax
import jax.numpy as jnp
from jax.experimental import pallas as pl
from jax.experimental.pallas import tpu as pltpu

_S = 4
_K = 1024
_D = 256
_T = 512


def _mlp_body(cb_ref, w1_ref, b1_ref, w2_ref, b2_ref,
              cbeff_ref, bytes_ref):
    for i in range(_S):
        cb = cb_ref[i]
        h = jnp.maximum(jnp.dot(cb, w1_ref[i]) + b1_ref[i], 0.0)
        cb_eff = jnp.dot(h, w2_ref[i]) + b2_ref[i]
        cbeff_ref[i] = cb_eff
        u = jax.lax.bitcast_convert_type(cb_eff, jnp.int32)
        for j in range(4):
            bytes_ref[i, j] = ((u >> (8 * j)) & 0xFF).astype(jnp.float32)


def _stage_body(r_ref, r2_ref, cb_ref, c2_ref, bytes_ref, q_ref):
    residual = r_ref[...]
    cb_eff = cb_ref[...]
    e = jax.lax.dot_general(residual, cb_eff,
                            (((1,), (1,)), ((), ())))
    d = (r2_ref[...] - 2.0 * e) + c2_ref[...]
    lanes = jax.lax.broadcasted_iota(jnp.int32, (_T, _K), 1)
    mn = jnp.min(d, axis=-1, keepdims=True)
    idx = jnp.min(jnp.where(d == mn, lanes, _K), axis=-1)
    onehot = (lanes == idx[:, None]).astype(jnp.float32)
    qb = [jax.lax.dot_general(onehot, bytes_ref[j],
                              (((1,), (0,)), ((), ())))
          for j in range(4)]
    qi = (qb[0].astype(jnp.int32)
          | (qb[1].astype(jnp.int32) << 8)
          | (qb[2].astype(jnp.int32) << 16)
          | (qb[3].astype(jnp.int32) << 24))
    q_ref[...] = jax.lax.bitcast_convert_type(qi, jnp.float32)


def kernel(z, codebooks, W1, b1, W2, b2):
    B, N, D = z.shape
    nt = B * N
    z2 = z.reshape(nt, D)

    cb_eff, planes = pl.pallas_call(
        _mlp_body,
        out_shape=[
            jax.ShapeDtypeStruct((_S, _K, _D), jnp.float32),
            jax.ShapeDtypeStruct((_S, 4, _K, _D), jnp.float32),
        ],
    )(codebooks, W1, b1.reshape(_S, 1, _D), W2, b2.reshape(_S, 1, _D))
    c2 = jnp.sum(cb_eff ** 2, axis=-1).reshape(_S, 1, _K)

    stage = pl.pallas_call(
        _stage_body,
        grid=(nt // _T,),
        in_specs=[
            pl.BlockSpec((_T, _D), lambda t: (t, 0)),
            pl.BlockSpec((_T, 1), lambda t: (t, 0)),
            pl.BlockSpec((_K, _D), lambda t: (0, 0)),
            pl.BlockSpec((1, _K), lambda t: (0, 0)),
            pl.BlockSpec((4, _K, _D), lambda t: (0, 0, 0)),
        ],
        out_specs=pl.BlockSpec((_T, _D), lambda t: (t, 0)),
        out_shape=jax.ShapeDtypeStruct((nt, D), jnp.float32),
    )

    residual = z
    total = jnp.zeros_like(z)
    for i in range(_S):
        r2 = jnp.sum(residual ** 2, axis=-1, keepdims=True)
        q = stage(residual.reshape(nt, D), r2.reshape(nt, 1),
                  cb_eff[i], c2[i], planes[i]).reshape(B, N, D)
        residual = residual - q
        total = total + q
    out = z + (total - z)
    return out


# speedup vs baseline: 1.1222x; 1.0223x over previous
import jax
import jax.numpy as jnp
from jax.experimental import pallas as pl
from jax.experimental.pallas import tpu as pltpu

_S = 4
_K = 1024
_D = 256
_T = 512


def _mlp_body(cb_ref, w1_ref, b1_ref, w2_ref, b2_ref,
              cbeff_ref, bytes_ref):
    for i in range(_S):
        cb = cb_ref[i]
        h = jnp.maximum(jnp.dot(cb, w1_ref[i]) + b1_ref[i], 0.0)
        cb_eff = jnp.dot(h, w2_ref[i]) + b2_ref[i]
        cbeff_ref[i] = cb_eff
        u = jax.lax.bitcast_convert_type(cb_eff, jnp.int32)
        for j in range(4):
            bytes_ref[i, j] = ((u >> (8 * j)) & 0xFF).astype(jnp.bfloat16)


def _stage_body(r_ref, r2_ref, cb_ref, c2_ref, bytes_ref, q_ref):
    residual = r_ref[...]
    cb_eff = cb_ref[...]
    e = jax.lax.dot_general(residual, cb_eff,
                            (((1,), (1,)), ((), ())))
    d = (r2_ref[...] - 2.0 * e) + c2_ref[...]
    lanes = jax.lax.broadcasted_iota(jnp.int32, (_T, _K), 1)
    mn = jnp.min(d, axis=-1, keepdims=True)
    idx = jnp.min(jnp.where(d == mn, lanes, _K), axis=-1)
    onehot = (lanes == idx[:, None]).astype(jnp.bfloat16)
    qb = [jax.lax.dot_general(onehot, bytes_ref[j],
                              (((1,), (0,)), ((), ())),
                              preferred_element_type=jnp.float32)
          for j in range(4)]
    qi = (qb[0].astype(jnp.int32)
          | (qb[1].astype(jnp.int32) << 8)
          | (qb[2].astype(jnp.int32) << 16)
          | (qb[3].astype(jnp.int32) << 24))
    q_ref[...] = jax.lax.bitcast_convert_type(qi, jnp.float32)


def kernel(z, codebooks, W1, b1, W2, b2):
    B, N, D = z.shape
    nt = B * N
    z2 = z.reshape(nt, D)

    cb_eff, planes = pl.pallas_call(
        _mlp_body,
        out_shape=[
            jax.ShapeDtypeStruct((_S, _K, _D), jnp.float32),
            jax.ShapeDtypeStruct((_S, 4, _K, _D), jnp.bfloat16),
        ],
    )(codebooks, W1, b1.reshape(_S, 1, _D), W2, b2.reshape(_S, 1, _D))
    c2 = jnp.sum(cb_eff ** 2, axis=-1).reshape(_S, 1, _K)

    stage = pl.pallas_call(
        _stage_body,
        grid=(nt // _T,),
        in_specs=[
            pl.BlockSpec((_T, _D), lambda t: (t, 0)),
            pl.BlockSpec((_T, 1), lambda t: (t, 0)),
            pl.BlockSpec((_K, _D), lambda t: (0, 0)),
            pl.BlockSpec((1, _K), lambda t: (0, 0)),
            pl.BlockSpec((4, _K, _D), lambda t: (0, 0, 0)),
        ],
        out_specs=pl.BlockSpec((_T, _D), lambda t: (t, 0)),
        out_shape=jax.ShapeDtypeStruct((nt, D), jnp.float32),
    )

    residual = z
    total = jnp.zeros_like(z)
    for i in range(_S):
        r2 = jnp.sum(residual ** 2, axis=-1, keepdims=True)
        q = stage(residual.reshape(nt, D), r2.reshape(nt, 1),
                  cb_eff[i], c2[i], planes[i]).reshape(B, N, D)
        residual = residual - q
        total = total + q
    out = z + (total - z)
    return out


# speedup vs baseline: 1.6077x; 1.4327x over previous
import jax
import jax.numpy as jnp
from jax.experimental import pallas as pl
from jax.experimental.pallas import tpu as pltpu

_S = 4
_K = 1024
_D = 256
_T = 512


def _mlp_body(cb_ref, w1_ref, b1_ref, w2_ref, b2_ref,
              cbeff_ref, bytes_ref):
    for i in range(_S):
        cb = cb_ref[i]
        h = jnp.maximum(jnp.dot(cb, w1_ref[i]) + b1_ref[i], 0.0)
        cb_eff = jnp.dot(h, w2_ref[i]) + b2_ref[i]
        cbeff_ref[i] = cb_eff
        u = jax.lax.bitcast_convert_type(cb_eff, jnp.int32)
        for j in range(4):
            bytes_ref[i, j] = ((u >> (8 * j)) & 0xFF).astype(jnp.bfloat16)


def _fused_body(z_ref, r20_ref, cb_ref, c2_ref, bytes_ref, out_ref):
    z = z_ref[...]
    lanes = jax.lax.broadcasted_iota(jnp.int32, (_T, _K), 1)
    residual = z
    total = jnp.zeros_like(z)
    for i in range(_S):
        if i == 0:
            r2 = r20_ref[...]
        else:
            r2 = jnp.sum(residual * residual, axis=-1, keepdims=True)
        e = jax.lax.dot_general(residual, cb_ref[i],
                                (((1,), (1,)), ((), ())))
        d = (r2 - 2.0 * e) + c2_ref[i]
        mn = jnp.min(d, axis=-1, keepdims=True)
        idx = jnp.min(jnp.where(d == mn, lanes, _K), axis=-1)
        onehot = (lanes == idx[:, None]).astype(jnp.bfloat16)
        qb = [jax.lax.dot_general(onehot, bytes_ref[i, j],
                                  (((1,), (0,)), ((), ())),
                                  preferred_element_type=jnp.float32)
              for j in range(4)]
        qi = (qb[0].astype(jnp.int32)
              | (qb[1].astype(jnp.int32) << 8)
              | (qb[2].astype(jnp.int32) << 16)
              | (qb[3].astype(jnp.int32) << 24))
        q = jax.lax.bitcast_convert_type(qi, jnp.float32)
        residual = residual - q
        total = total + q
    out_ref[...] = z + (total - z)


def kernel(z, codebooks, W1, b1, W2, b2):
    B, N, D = z.shape
    nt = B * N

    cb_eff, planes = pl.pallas_call(
        _mlp_body,
        out_shape=[
            jax.ShapeDtypeStruct((_S, _K, _D), jnp.float32),
            jax.ShapeDtypeStruct((_S, 4, _K, _D), jnp.bfloat16),
        ],
    )(codebooks, W1, b1.reshape(_S, 1, _D), W2, b2.reshape(_S, 1, _D))
    c2 = jnp.sum(cb_eff ** 2, axis=-1).reshape(_S, 1, _K)
    r20 = jnp.sum(z ** 2, axis=-1).reshape(nt, 1)

    out = pl.pallas_call(
        _fused_body,
        grid=(nt // _T,),
        in_specs=[
            pl.BlockSpec((_T, _D), lambda t: (t, 0)),
            pl.BlockSpec((_T, 1), lambda t: (t, 0)),
            pl.BlockSpec((_S, _K, _D), lambda t: (0, 0, 0)),
            pl.BlockSpec((_S, 1, _K), lambda t: (0, 0, 0)),
            pl.BlockSpec((_S, 4, _K, _D), lambda t: (0, 0, 0, 0)),
        ],
        out_specs=pl.BlockSpec((_T, _D), lambda t: (t, 0)),
        out_shape=jax.ShapeDtypeStruct((nt, D), jnp.float32),
    )(z.reshape(nt, D), r20, cb_eff, c2, planes)
    return out.reshape(B, N, D)
